# baseline (speedup 1.0000x reference)
import functools

import jax
import jax.numpy as jnp
from jax import lax
from jax.experimental import pallas as pl
from jax.experimental.pallas import tpu as pltpu
from jax.experimental.pallas import tpu_sc as plsc

B = 64
N = 50000
E = 800000
D_F = 128
D_E = 16
D_G = 128
D_OUT = 128

NC = 2
NS = 16
NW = NC * NS

EW = 128 // D_E
EROWS = E // EW

S = 640
NODE_CHUNKS = -(-N // S)
EDGE_CHUNKS = -(-EROWS // S)
NODE_TAIL = N - (NODE_CHUNKS - 1) * S
EDGE_TAIL = EROWS - (EDGE_CHUNKS - 1) * S
NODE_PAD = NODE_CHUNKS * S
EDGE_PAD = EDGE_CHUNKS * S * EW
ACC_ROWS = B + 1

_mesh = plsc.VectorSubcoreMesh(
    core_axis_name="c", subcore_axis_name="s", num_cores=NC, num_subcores=NS
)


@functools.partial(
    pl.kernel,
    out_type=(
        jax.ShapeDtypeStruct((NW, B, D_F), jnp.float32),
        jax.ShapeDtypeStruct((NW, B, D_E), jnp.float32),
    ),
    mesh=_mesh,
    compiler_params=pltpu.CompilerParams(needs_layout_passes=False),
    scratch_types=[
        pltpu.VMEM((S, 128), jnp.float32),
        pltpu.VMEM((S,), jnp.int32),
        pltpu.VMEM((S * EW,), jnp.int32),
        pltpu.VMEM((ACC_ROWS, D_F), jnp.float32),
        pltpu.VMEM((ACC_ROWS, D_E), jnp.float32),
    ],
)
def _sc_segsum(
    node_hbm, nidx_hbm, edge_hbm, eidx_hbm,
    npart_hbm, epart_hbm,
    rows_v, nidx_v, eidx_v, nacc_v, eacc_v,
):
    cid = lax.axis_index("c")
    sid = lax.axis_index("s")
    wid = cid * NS + sid

    fzero = jnp.zeros((16,), jnp.float32)
    iota = lax.iota(jnp.int32, 16)

    def zrow(r, carry):
        for g in range(D_F // 16):
            nacc_v[r, pl.ds(g * 16, 16)] = fzero
        eacc_v[r, pl.ds(0, 16)] = fzero
        return carry

    lax.fori_loop(0, ACC_ROWS, zrow, 0)

    def bcast_lane(v, lane):
        return jnp.sum(jnp.where(iota == lane, v, 0))

    def node_group(t, carry):
        idxv = nidx_v[pl.ds(t * 16, 16)]
        lo = jnp.min(idxv)
        hi = jnp.max(idxv)

        @pl.when(lo == hi)
        def _():
            for g in range(D_F // 16):
                acc = rows_v[t * 16, pl.ds(g * 16, 16)]
                for r in range(1, 16):
                    acc += rows_v[t * 16 + r, pl.ds(g * 16, 16)]
                plsc.addupdate_scatter(
                    nacc_v, [jnp.full((16,), lo, jnp.int32), g * 16 + iota], acc)

        @pl.when(lo != hi)
        def _():
            for r in range(16):
                seg = bcast_lane(idxv, r)
                rowi = jnp.full((16,), seg, jnp.int32)
                for g in range(D_F // 16):
                    plsc.addupdate_scatter(
                        nacc_v, [rowi, g * 16 + iota],
                        rows_v[t * 16 + r, pl.ds(g * 16, 16)])

        return carry

    def edge_group(t, carry):
        idxv = eidx_v[pl.ds(t * 16, 16)]
        lo = jnp.min(idxv)
        hi = jnp.max(idxv)

        @pl.when(lo == hi)
        def _():
            acc = rows_v[t * 2, pl.ds(0, 16)]
            for j in range(1, 16):
                acc += rows_v[t * 2 + j // 8, pl.ds((j % 8) * 16, 16)]
            plsc.addupdate_scatter(
                eacc_v, [jnp.full((16,), lo, jnp.int32), iota], acc)

        @pl.when(lo != hi)
        def _():
            for j in range(16):
                seg = bcast_lane(idxv, j)
                plsc.addupdate_scatter(
                    eacc_v, [jnp.full((16,), seg, jnp.int32), iota],
                    rows_v[t * 2 + j // 8, pl.ds((j % 8) * 16, 16)])

        return carry

    def seg_loop(attr_hbm, idx_hbm, idx_v, ids_per_row, group_fn,
                 nchunks, tail):
        def body(k, carry):
            c = wid + k * NW

            @pl.when(c < nchunks)
            def _():
                pltpu.sync_copy(
                    idx_hbm.at[pl.ds(c * S * ids_per_row, S * ids_per_row)],
                    idx_v)
                if tail == S:
                    pltpu.sync_copy(attr_hbm.at[pl.ds(c * S, S)], rows_v)
                else:
                    @pl.when(c < nchunks - 1)
                    def _():
                        pltpu.sync_copy(attr_hbm.at[pl.ds(c * S, S)], rows_v)

                    @pl.when(c == nchunks - 1)
                    def _():
                        pltpu.sync_copy(
                            attr_hbm.at[pl.ds(c * S, tail)],
                            rows_v.at[pl.ds(0, tail)])

                lax.fori_loop(0, S * ids_per_row // 16, group_fn, 0)

            return carry

        lax.fori_loop(0, -(-nchunks // NW), body, 0)

    seg_loop(node_hbm, nidx_hbm, nidx_v, 1, node_group,
             NODE_CHUNKS, NODE_TAIL)
    seg_loop(edge_hbm, eidx_hbm, eidx_v, EW, edge_group,
             EDGE_CHUNKS, EDGE_TAIL)

    pltpu.sync_copy(nacc_v.at[pl.ds(0, B)], npart_hbm.at[wid])
    pltpu.sync_copy(eacc_v.at[pl.ds(0, B)], epart_hbm.at[wid])


def _finish_body(g_ref, np_ref, ep_ref, w_ref, b_ref, o_ref):
    nacc = jnp.sum(np_ref[...], axis=0)
    eacc = jnp.sum(ep_ref[...], axis=0)
    out = jnp.dot(g_ref[...], w_ref[pl.ds(0, D_G), :],
                  preferred_element_type=jnp.float32)
    out += jnp.dot(nacc, w_ref[pl.ds(D_G, D_F), :],
                   preferred_element_type=jnp.float32)
    out += jnp.dot(eacc, w_ref[pl.ds(D_G + D_F, D_E), :],
                   preferred_element_type=jnp.float32)
    o_ref[...] = out + b_ref[...]


_finish = pl.pallas_call(
    _finish_body,
    out_shape=jax.ShapeDtypeStruct((B, D_OUT), jnp.float32),
)


def kernel(global_attr, node_attr, edge_attr, edges, node_idx, edge_idx, W, b):
    del edges
    nidx = node_idx.astype(jnp.int32)
    eidx = edge_idx.astype(jnp.int32)
    nidx_p = jnp.concatenate([nidx, jnp.full((NODE_PAD - N,), B, jnp.int32)])
    eidx_p = jnp.concatenate([eidx, jnp.full((EDGE_PAD - E,), B, jnp.int32)])
    edge_wide = edge_attr.reshape(EROWS, 128)

    npart, epart = _sc_segsum(node_attr, nidx_p, edge_wide, eidx_p)
    return _finish(global_attr, npart, epart, W, b.reshape(1, D_OUT))


# speedup vs baseline: 1.0249x; 1.0249x over previous
import functools

import jax
import jax.numpy as jnp
from jax import lax
from jax.experimental import pallas as pl
from jax.experimental.pallas import tpu as pltpu
from jax.experimental.pallas import tpu_sc as plsc

B = 64
N = 50000
E = 800000
D_F = 128
D_E = 16
D_G = 128
D_OUT = 128

NC = 2
NS = 16
NW = NC * NS

EW = 128 // D_E
EROWS = E // EW

S = 640
NODE_CHUNKS = -(-N // S)
EDGE_CHUNKS = -(-EROWS // S)
NODE_TAIL = N - (NODE_CHUNKS - 1) * S
EDGE_TAIL = EROWS - (EDGE_CHUNKS - 1) * S
NODE_PAD = NODE_CHUNKS * S
EDGE_PAD = EDGE_CHUNKS * S * EW
ACC_ROWS = B + 1

_mesh = plsc.VectorSubcoreMesh(
    core_axis_name="c", subcore_axis_name="s", num_cores=NC, num_subcores=NS
)


@functools.partial(
    pl.kernel,
    out_type=(
        jax.ShapeDtypeStruct((NW, B, D_F), jnp.float32),
        jax.ShapeDtypeStruct((NW, B, D_E), jnp.float32),
    ),
    mesh=_mesh,
    compiler_params=pltpu.CompilerParams(needs_layout_passes=False),
    scratch_types=[
        pltpu.VMEM((S, 128), jnp.float32),
        pltpu.VMEM((S,), jnp.int32),
        pltpu.VMEM((S * EW,), jnp.int32),
        pltpu.VMEM((ACC_ROWS, D_F), jnp.float32),
        pltpu.VMEM((ACC_ROWS, D_E), jnp.float32),
    ],
)
def _sc_segsum(
    node_hbm, nidx_hbm, edge_hbm, eidx_hbm,
    npart_hbm, epart_hbm,
    rows_v, nidx_v, eidx_v, nacc_v, eacc_v,
):
    cid = lax.axis_index("c")
    sid = lax.axis_index("s")
    wid = cid * NS + sid

    fzero = jnp.zeros((16,), jnp.float32)
    iota = lax.iota(jnp.int32, 16)

    def zrow(r, carry):
        for g in range(D_F // 16):
            nacc_v[r, pl.ds(g * 16, 16)] = fzero
        eacc_v[r, pl.ds(0, 16)] = fzero
        return carry

    lax.fori_loop(0, ACC_ROWS, zrow, 0)

    def bcast_lane(v, lane):
        return jnp.sum(jnp.where(iota == lane, v, 0))

    def node_group(t, carry):
        idxv = nidx_v[pl.ds(t * 16, 16)]
        lo = jnp.min(idxv)
        hi = jnp.max(idxv)

        @pl.when(lo == hi)
        def _():
            rowi = jnp.full((16,), lo, jnp.int32)
            for g in range(D_F // 16):
                vals = [rows_v[t * 16 + r, pl.ds(g * 16, 16)]
                        for r in range(16)]
                while len(vals) > 1:
                    vals = [vals[i] + vals[i + 1]
                            for i in range(0, len(vals) - 1, 2)] + (
                        [vals[-1]] if len(vals) % 2 else [])
                plsc.addupdate_scatter(nacc_v, [rowi, g * 16 + iota], vals[0])

        @pl.when(lo != hi)
        def _():
            for r in range(16):
                seg = bcast_lane(idxv, r)
                rowi = jnp.full((16,), seg, jnp.int32)
                for g in range(D_F // 16):
                    plsc.addupdate_scatter(
                        nacc_v, [rowi, g * 16 + iota],
                        rows_v[t * 16 + r, pl.ds(g * 16, 16)])

        return carry

    def edge_group(t, carry):
        idxv = eidx_v[pl.ds(t * 16, 16)]
        lo = jnp.min(idxv)
        hi = jnp.max(idxv)

        @pl.when(lo == hi)
        def _():
            vals = [rows_v[t * 2 + j // 8, pl.ds((j % 8) * 16, 16)]
                    for j in range(16)]
            while len(vals) > 1:
                vals = [vals[i] + vals[i + 1]
                        for i in range(0, len(vals) - 1, 2)] + (
                    [vals[-1]] if len(vals) % 2 else [])
            plsc.addupdate_scatter(
                eacc_v, [jnp.full((16,), lo, jnp.int32), iota], vals[0])

        @pl.when(lo != hi)
        def _():
            for j in range(16):
                seg = bcast_lane(idxv, j)
                plsc.addupdate_scatter(
                    eacc_v, [jnp.full((16,), seg, jnp.int32), iota],
                    rows_v[t * 2 + j // 8, pl.ds((j % 8) * 16, 16)])

        return carry

    def seg_loop(attr_hbm, idx_hbm, idx_v, ids_per_row, group_fn,
                 nchunks, tail):
        def body(k, carry):
            c = wid + k * NW

            @pl.when(c < nchunks)
            def _():
                pltpu.sync_copy(
                    idx_hbm.at[pl.ds(c * S * ids_per_row, S * ids_per_row)],
                    idx_v)
                if tail == S:
                    pltpu.sync_copy(attr_hbm.at[pl.ds(c * S, S)], rows_v)
                else:
                    @pl.when(c < nchunks - 1)
                    def _():
                        pltpu.sync_copy(attr_hbm.at[pl.ds(c * S, S)], rows_v)

                    @pl.when(c == nchunks - 1)
                    def _():
                        pltpu.sync_copy(
                            attr_hbm.at[pl.ds(c * S, tail)],
                            rows_v.at[pl.ds(0, tail)])

                lax.fori_loop(0, S * ids_per_row // 16, group_fn, 0)

            return carry

        lax.fori_loop(0, -(-nchunks // NW), body, 0)

    seg_loop(node_hbm, nidx_hbm, nidx_v, 1, node_group,
             NODE_CHUNKS, NODE_TAIL)
    seg_loop(edge_hbm, eidx_hbm, eidx_v, EW, edge_group,
             EDGE_CHUNKS, EDGE_TAIL)

    pltpu.sync_copy(nacc_v.at[pl.ds(0, B)], npart_hbm.at[wid])
    pltpu.sync_copy(eacc_v.at[pl.ds(0, B)], epart_hbm.at[wid])


def _finish_body(g_ref, np_ref, ep_ref, w_ref, b_ref, o_ref):
    nacc = jnp.sum(np_ref[...], axis=0)
    eacc = jnp.sum(ep_ref[...], axis=0)
    out = jnp.dot(g_ref[...], w_ref[pl.ds(0, D_G), :],
                  preferred_element_type=jnp.float32)
    out += jnp.dot(nacc, w_ref[pl.ds(D_G, D_F), :],
                   preferred_element_type=jnp.float32)
    out += jnp.dot(eacc, w_ref[pl.ds(D_G + D_F, D_E), :],
                   preferred_element_type=jnp.float32)
    o_ref[...] = out + b_ref[...]


_finish = pl.pallas_call(
    _finish_body,
    out_shape=jax.ShapeDtypeStruct((B, D_OUT), jnp.float32),
)


def kernel(global_attr, node_attr, edge_attr, edges, node_idx, edge_idx, W, b):
    del edges
    nidx = node_idx.astype(jnp.int32)
    eidx = edge_idx.astype(jnp.int32)
    nidx_p = jnp.concatenate([nidx, jnp.full((NODE_PAD - N,), B, jnp.int32)])
    eidx_p = jnp.concatenate([eidx, jnp.full((EDGE_PAD - E,), B, jnp.int32)])
    edge_wide = edge_attr.reshape(EROWS, 128)

    npart, epart = _sc_segsum(node_attr, nidx_p, edge_wide, eidx_p)
    return _finish(global_attr, npart, epart, W, b.reshape(1, D_OUT))
